# Initial kernel scaffold; baseline (speedup 1.0000x reference)
import jax
import jax.numpy as jnp
from jax.experimental import pallas as pl


def kernel(node_features, edge_index, edge_weights, W0, b0, gamma0, beta0, W1, b1, gamma1, beta1, W2, b2, gamma2, beta2):
    raise NotImplementedError("write your pallas kernel here")


# speedup vs baseline: 4.1263x; 4.1263x over previous
import functools

import jax
import jax.numpy as jnp
from jax import lax
from jax.experimental import pallas as pl
from jax.experimental.pallas import tpu as pltpu
from jax.experimental.pallas import tpu_sc as plsc

N = 10000
NP = 10240
E = 320000
D = 128
EPS = 1e-5

NC = 2
NS = 16
NW = NC * NS
EPW = E // NW
CH = 80
NCH = EPW // CH
RPT = NP // NS

_mesh = plsc.VectorSubcoreMesh(core_axis_name="c", subcore_axis_name="s")


@functools.partial(
    pl.kernel,
    out_type=jax.ShapeDtypeStruct((NC, NP, D), jnp.float32),
    mesh=_mesh,
    scratch_types=[
        pltpu.VMEM((CH,), jnp.int32),
        pltpu.VMEM((CH,), jnp.int32),
        pltpu.VMEM((CH,), jnp.float32),
        pltpu.VMEM((CH, D), jnp.float32),
        pltpu.VMEM_SHARED((NP, D), jnp.float32),
        pltpu.SemaphoreType.DMA,
    ],
)
def _sc_aggregate(h_hbm, src_hbm, dst_hbm, w_hbm, zero_hbm, out_hbm,
                  src_v, dst_v, w_v, rows_v, aggr_s, sem):
    c = lax.axis_index("c")
    s = lax.axis_index("s")
    wid = c * NS + s

    pltpu.sync_copy(zero_hbm.at[pl.ds(s * RPT, RPT)],
                    aggr_s.at[pl.ds(s * RPT, RPT)])
    plsc.subcore_barrier()

    def chunk_body(k, carry):
        base = wid * EPW + k * CH
        pltpu.sync_copy(src_hbm.at[pl.ds(base, CH)], src_v)
        pltpu.sync_copy(dst_hbm.at[pl.ds(base, CH)], dst_v)
        pltpu.sync_copy(w_hbm.at[pl.ds(base, CH)], w_v)
        pltpu.async_copy(h_hbm.at[src_v], rows_v, sem).wait()

        def group_body(g, carry2):
            e0 = g * 16
            w16 = w_v[pl.ds(e0, 16)]
            for i in range(16):
                wb = jnp.full((16,), w16[i], dtype=jnp.float32)
                for j in range(D // 16):
                    sl = pl.ds(j * 16, 16)
                    rows_v[e0 + i, sl] = rows_v[e0 + i, sl] * wb
            return carry2

        lax.fori_loop(0, CH // 16, group_body, 0)
        pltpu.sync_copy(rows_v, aggr_s.at[dst_v], add=True)
        return carry

    lax.fori_loop(0, NCH, chunk_body, 0)

    plsc.subcore_barrier()
    pltpu.sync_copy(aggr_s.at[pl.ds(s * RPT, RPT)],
                    out_hbm.at[c, pl.ds(s * RPT, RPT)])


_BN = 2048


def _tc_body(p_ref, w_ref, b_ref, o_ref):
    x = p_ref[0] + p_ref[1]
    y = jnp.dot(x, w_ref[...], preferred_element_type=jnp.float32)
    o_ref[...] = jnp.maximum(y + b_ref[...], 0.0)


def _tc_update(part, wt, bias):
    return pl.pallas_call(
        _tc_body,
        out_shape=jax.ShapeDtypeStruct((NP, D), jnp.float32),
        grid=(NP // _BN,),
        in_specs=[
            pl.BlockSpec((NC, _BN, D), lambda i: (0, i, 0)),
            pl.BlockSpec((D, D), lambda i: (0, 0)),
            pl.BlockSpec((1, D), lambda i: (0, 0)),
        ],
        out_specs=pl.BlockSpec((_BN, D), lambda i: (i, 0)),
    )(part, wt, bias)


def kernel(node_features, edge_index, edge_weights,
           W0, b0, gamma0, beta0,
           W1, b1, gamma1, beta1,
           W2, b2, gamma2, beta2):
    src = edge_index[0]
    dst = edge_index[1]
    zero = jnp.zeros((NP, D), jnp.float32)
    scale = 1.0 / jnp.sqrt(jnp.float32(1.0) + EPS)
    h = jnp.concatenate([node_features, jnp.zeros((NP - N, D), jnp.float32)], axis=0)
    for W, b, g, bt in ((W0, b0, gamma0, beta0),
                        (W1, b1, gamma1, beta1),
                        (W2, b2, gamma2, beta2)):
        geff = g * scale
        wt = (W * geff[:, None]).T
        bias = (b * geff + bt)[None, :]
        part = _sc_aggregate(h, src, dst, edge_weights, zero)
        h = _tc_update(part, wt, bias)
    return h[:N]


# speedup vs baseline: 7.8752x; 1.9086x over previous
import functools

import jax
import jax.numpy as jnp
from jax import lax
from jax.experimental import pallas as pl
from jax.experimental.pallas import tpu as pltpu
from jax.experimental.pallas import tpu_sc as plsc

N = 10000
NP = 10112
E = 320000
D = 128
EPS = 1e-5

NC = 2
NS = 16
NW = NC * NS
EPW = E // NW
CH = 80
NCH = EPW // CH
RPT = NP // NS

_mesh = plsc.VectorSubcoreMesh(core_axis_name="c", subcore_axis_name="s")


@functools.partial(
    pl.kernel,
    out_type=jax.ShapeDtypeStruct((NC, NP, D), jnp.float32),
    mesh=_mesh,
    scratch_types=[
        pltpu.VMEM((CH,), jnp.int32),
        pltpu.VMEM((CH,), jnp.int32),
        pltpu.VMEM((CH,), jnp.int32),
        pltpu.VMEM((CH,), jnp.int32),
        pltpu.VMEM((CH,), jnp.float32),
        pltpu.VMEM((CH,), jnp.float32),
        pltpu.VMEM((CH,), jnp.int32),
        pltpu.VMEM((CH,), jnp.int32),
        pltpu.VMEM((CH, D), jnp.float32),
        pltpu.VMEM((CH, D), jnp.float32),
        pltpu.VMEM_SHARED((NP, D), jnp.float32),
        pltpu.SemaphoreType.DMA,
        pltpu.SemaphoreType.DMA,
        pltpu.SemaphoreType.DMA,
        pltpu.SemaphoreType.DMA,
        pltpu.SemaphoreType.DMA,
        pltpu.SemaphoreType.DMA,
    ],
)
def _sc_aggregate(h_hbm, src_hbm, dst_hbm, w_hbm, zero_hbm, out_hbm,
                  src0, src1, dst0, dst1, w0, w1, dsts0, dsts1,
                  rows0, rows1, aggr_s,
                  semi0, semi1, semg0, semg1, sems0, sems1):
    c = lax.axis_index("c")
    s = lax.axis_index("s")
    ebase = (c * NS + s) * EPW

    srcb = (src0, src1)
    dstb = (dst0, dst1)
    wb = (w0, w1)
    dstsb = (dsts0, dsts1)
    rowsb = (rows0, rows1)
    semi = (semi0, semi1)
    semg = (semg0, semg1)
    sems = (sems0, sems1)

    pltpu.sync_copy(zero_hbm.at[pl.ds(s * RPT, RPT)],
                    aggr_s.at[pl.ds(s * RPT, RPT)])

    def idx_load(k, b):
        base = ebase + k * CH
        pltpu.async_copy(src_hbm.at[pl.ds(base, CH)], srcb[b], semi[b])
        pltpu.async_copy(dst_hbm.at[pl.ds(base, CH)], dstb[b], semi[b])
        pltpu.async_copy(w_hbm.at[pl.ds(base, CH)], wb[b], semi[b])

    def idx_wait(b):
        pltpu.make_async_copy(src_hbm.at[pl.ds(0, CH)], srcb[b], semi[b]).wait()
        pltpu.make_async_copy(dst_hbm.at[pl.ds(0, CH)], dstb[b], semi[b]).wait()
        pltpu.make_async_copy(w_hbm.at[pl.ds(0, CH)], wb[b], semi[b]).wait()

    def gather_start(b):
        pltpu.async_copy(h_hbm.at[srcb[b]], rowsb[b], semg[b])

    def gather_wait(b):
        pltpu.make_async_copy(h_hbm.at[srcb[b]], rowsb[b], semg[b]).wait()

    def scale_and_copy(b):
        def gbody(g, carry):
            e0 = g * 16
            sl16 = pl.ds(e0, 16)
            w16 = wb[b][sl16]
            dstsb[b][sl16] = dstb[b][sl16]
            for i in range(16):
                wv = jnp.full((16,), w16[i], dtype=jnp.float32)
                for j in range(D // 16):
                    sl = pl.ds(j * 16, 16)
                    rowsb[b][e0 + i, sl] = rowsb[b][e0 + i, sl] * wv
            return carry
        lax.fori_loop(0, CH // 16, gbody, 0)

    def scatter_start(b):
        pltpu.async_copy(rowsb[b], aggr_s.at[dstsb[b]], sems[b], add=True)

    def scatter_wait(b):
        pltpu.make_async_copy(rowsb[b], aggr_s.at[dstsb[b]], sems[b]).wait()

    def chunk_step(k, b, nb, prefetch):
        gather_wait(b)
        scale_and_copy(b)
        scatter_start(b)

        @pl.when(k < NCH - 2)
        def _():
            idx_load(k + 2, b)

        if prefetch:
            @pl.when(k > 0)
            def _():
                scatter_wait(nb)
            idx_wait(nb)
            gather_start(nb)

    idx_load(0, 0)
    idx_load(1, 1)
    idx_wait(0)
    plsc.subcore_barrier()
    gather_start(0)

    def group(g, carry):
        for b in range(2):
            k = g * 2 + b
            chunk_step(k, b, 1 - b, prefetch=True)
        return carry

    lax.fori_loop(0, (NCH - 1) // 2, group, 0)

    chunk_step(NCH - 1, (NCH - 1) % 2, NCH % 2, prefetch=False)
    scatter_wait(0)
    scatter_wait(1)
    plsc.subcore_barrier()
    pltpu.sync_copy(aggr_s.at[pl.ds(s * RPT, RPT)],
                    out_hbm.at[c, pl.ds(s * RPT, RPT)])


_BN = 2528


def _tc_body(p_ref, w_ref, b_ref, o_ref):
    x = p_ref[0] + p_ref[1]
    y = jnp.dot(x, w_ref[...], preferred_element_type=jnp.float32)
    o_ref[...] = jnp.maximum(y + b_ref[...], 0.0)


def _tc_update(part, wt, bias):
    return pl.pallas_call(
        _tc_body,
        out_shape=jax.ShapeDtypeStruct((NP, D), jnp.float32),
        grid=(NP // _BN,),
        in_specs=[
            pl.BlockSpec((NC, _BN, D), lambda i: (0, i, 0)),
            pl.BlockSpec((D, D), lambda i: (0, 0)),
            pl.BlockSpec((1, D), lambda i: (0, 0)),
        ],
        out_specs=pl.BlockSpec((_BN, D), lambda i: (i, 0)),
    )(part, wt, bias)


def kernel(node_features, edge_index, edge_weights,
           W0, b0, gamma0, beta0,
           W1, b1, gamma1, beta1,
           W2, b2, gamma2, beta2):
    src = edge_index[0]
    dst = edge_index[1]
    zero = jnp.zeros((NP, D), jnp.float32)
    scale = 1.0 / jnp.sqrt(jnp.float32(1.0) + EPS)
    h = jnp.concatenate(
        [node_features, jnp.zeros((NP - N, D), jnp.float32)], axis=0)
    for W, b, g, bt in ((W0, b0, gamma0, beta0),
                        (W1, b1, gamma1, beta1),
                        (W2, b2, gamma2, beta2)):
        geff = g * scale
        wt = (W * geff[:, None]).T
        bias = (b * geff + bt)[None, :]
        part = _sc_aggregate(h, src, dst, edge_weights, zero)
        h = _tc_update(part, wt, bias)
    return h[:N]
